# Initial kernel scaffold; baseline (speedup 1.0000x reference)
import jax
import jax.numpy as jnp
from jax.experimental import pallas as pl


def kernel(x):
    raise NotImplementedError("write your pallas kernel here")


# speedup vs baseline: 29.1288x; 29.1288x over previous
import numpy as np
import jax
import jax.numpy as jnp
from jax.experimental import pallas as pl
from jax.experimental.pallas import tpu as pltpu

T = 2048
FREQ = 1025
FP = 1152
K = 32
WIN = 64


def _build_consts():
    t = np.arange(T, dtype=np.float64)
    f = np.arange(FREQ, dtype=np.float64)
    ang = 2.0 * np.pi * np.outer(f, t) / T
    cos, sin = np.cos(ang), np.sin(ang)
    fc = np.zeros((FP, T)); fc[:FREQ] = cos
    fs = np.zeros((FP, T)); fs[:FREQ] = -sin
    w = np.full(FREQ, 2.0); w[0] = 1.0; w[-1] = 1.0
    cinv = cos.T * (w / T)
    sinv = -sin.T * (w / T)
    win = np.hamming(WIN)
    u = np.arange(T + WIN)
    src = np.where(u < 32, 31 - u, np.where(u < T + 32, u - 32, 2 * T + 31 - u))
    M = np.zeros((T, T))
    rows = np.arange(T)
    for j in range(WIN):
        M[rows, src[rows + j]] += win[j] / win.sum()
    gr = np.zeros((T, FP)); gr[:, :FREQ] = M @ cinv
    gi = np.zeros((T, FP)); gi[:, :FREQ] = M @ sinv
    return (fc.astype(np.float32), fs.astype(np.float32),
            gr.astype(np.float32), gi.astype(np.float32))


_FC, _FS, _GR, _GI = _build_consts()

_BM = 384
_BN = 512
_BT = 512


_PREC = jax.lax.Precision.HIGHEST


def _fwd_kernel(fc_ref, fs_ref, xm_ref, xr_ref, xi_ref, m2_ref):
    xm = xm_ref[...]
    xr = jnp.dot(fc_ref[...], xm, preferred_element_type=jnp.float32,
                 precision=_PREC)
    xi = jnp.dot(fs_ref[...], xm, preferred_element_type=jnp.float32,
                 precision=_PREC)
    xr_ref[...] = xr
    xi_ref[...] = xi
    m2_ref[...] = xr * xr + xi * xi


def _mask_kernel(m2_ref, xr_ref, xi_ref, xrm_ref, xim_ref):
    bits = pltpu.bitcast(m2_ref[...], jnp.int32)
    prefix = jnp.zeros((1, bits.shape[1]), jnp.int32)
    for b in range(30, -1, -1):
        cand = prefix | (1 << b)
        cnt = jnp.sum((bits >= cand).astype(jnp.int32), axis=0, keepdims=True)
        prefix = jnp.where(cnt >= K, cand, prefix)
    mask = bits >= prefix
    xrm_ref[...] = jnp.where(mask, xr_ref[...], 0.0)
    xim_ref[...] = jnp.where(mask, xi_ref[...], 0.0)


def _inv_kernel(gr_ref, gi_ref, xrm_ref, xim_ref, out_ref):
    out_ref[...] = (
        jnp.dot(gr_ref[...], xrm_ref[...], preferred_element_type=jnp.float32,
                precision=_PREC)
        + jnp.dot(gi_ref[...], xim_ref[...], preferred_element_type=jnp.float32,
                  precision=_PREC))


def kernel(x):
    B, Tt, C = x.shape
    N = B * C
    xm = jnp.moveaxis(x, 0, 1).reshape(Tt, N)

    f32 = jnp.float32
    xr, xi, m2 = pl.pallas_call(
        _fwd_kernel,
        grid=(FP // _BM, N // _BN),
        in_specs=[
            pl.BlockSpec((_BM, T), lambda i, j: (i, 0)),
            pl.BlockSpec((_BM, T), lambda i, j: (i, 0)),
            pl.BlockSpec((T, _BN), lambda i, j: (0, j)),
        ],
        out_specs=[
            pl.BlockSpec((_BM, _BN), lambda i, j: (i, j)),
            pl.BlockSpec((_BM, _BN), lambda i, j: (i, j)),
            pl.BlockSpec((_BM, _BN), lambda i, j: (i, j)),
        ],
        out_shape=[jax.ShapeDtypeStruct((FP, N), f32)] * 3,
    )(jnp.asarray(_FC), jnp.asarray(_FS), xm)

    xrm, xim = pl.pallas_call(
        _mask_kernel,
        grid=(N // _BN,),
        in_specs=[pl.BlockSpec((FP, _BN), lambda j: (0, j))] * 3,
        out_specs=[pl.BlockSpec((FP, _BN), lambda j: (0, j))] * 2,
        out_shape=[jax.ShapeDtypeStruct((FP, N), f32)] * 2,
    )(m2, xr, xi)

    out = pl.pallas_call(
        _inv_kernel,
        grid=(Tt // _BT, N // _BN),
        in_specs=[
            pl.BlockSpec((_BT, FP), lambda i, j: (i, 0)),
            pl.BlockSpec((_BT, FP), lambda i, j: (i, 0)),
            pl.BlockSpec((FP, _BN), lambda i, j: (0, j)),
            pl.BlockSpec((FP, _BN), lambda i, j: (0, j)),
        ],
        out_specs=pl.BlockSpec((_BT, _BN), lambda i, j: (i, j)),
        out_shape=jax.ShapeDtypeStruct((Tt, N), f32),
    )(jnp.asarray(_GR), jnp.asarray(_GI), xrm, xim)

    return jnp.moveaxis(out.reshape(Tt, B, C), 0, 1)


# speedup vs baseline: 35.6930x; 1.2253x over previous
import numpy as np
import jax
import jax.numpy as jnp
from jax.experimental import pallas as pl
from jax.experimental.pallas import tpu as pltpu

T = 2048
FREQ = 1025
FP = 1152
K = 32
WIN = 64


def _build_consts():
    t = np.arange(T, dtype=np.float64)
    f = np.arange(FREQ, dtype=np.float64)
    ang = 2.0 * np.pi * np.outer(f, t) / T
    cos, sin = np.cos(ang), np.sin(ang)
    fc = np.zeros((FP, T)); fc[:FREQ] = cos
    fs = np.zeros((FP, T)); fs[:FREQ] = -sin
    w = np.full(FREQ, 2.0); w[0] = 1.0; w[-1] = 1.0
    cinv = cos.T * (w / T)
    sinv = -sin.T * (w / T)
    win = np.hamming(WIN)
    u = np.arange(T + WIN)
    src = np.where(u < 32, 31 - u, np.where(u < T + 32, u - 32, 2 * T + 31 - u))
    M = np.zeros((T, T))
    rows = np.arange(T)
    for j in range(WIN):
        M[rows, src[rows + j]] += win[j] / win.sum()
    gr = np.zeros((T, FP)); gr[:, :FREQ] = M @ cinv
    gi = np.zeros((T, FP)); gi[:, :FREQ] = M @ sinv
    return (fc.astype(np.float32), fs.astype(np.float32),
            gr.astype(np.float32), gi.astype(np.float32))


_FC, _FS, _GR, _GI = _build_consts()

_BM = 384
_BN = 512
_BT = 512


_PREC = jax.lax.Precision.HIGHEST


def _split_bf16(a32):
    import ml_dtypes
    hi = a32.astype(ml_dtypes.bfloat16)
    lo = (a32 - hi.astype(np.float32)).astype(ml_dtypes.bfloat16)
    return hi, lo


_GRH, _GRL = _split_bf16(_GR)
_GIH, _GIL = _split_bf16(_GI)


def _fwd_kernel(fc_ref, fs_ref, xm_ref, xr_ref, xi_ref, m2_ref):
    xm = xm_ref[...]
    xr = jnp.dot(fc_ref[...], xm, preferred_element_type=jnp.float32,
                 precision=_PREC)
    xi = jnp.dot(fs_ref[...], xm, preferred_element_type=jnp.float32,
                 precision=_PREC)
    xr_ref[...] = xr
    xi_ref[...] = xi
    m2_ref[...] = xr * xr + xi * xi


def _mask_kernel(m2_ref, xr_ref, xi_ref, xrm_ref, xim_ref):
    bits = pltpu.bitcast(m2_ref[...], jnp.int32)
    prefix = jnp.zeros((1, bits.shape[1]), jnp.int32)
    for b in range(30, -1, -1):
        cand = prefix | (1 << b)
        cnt = jnp.sum((bits >= cand).astype(jnp.int32), axis=0, keepdims=True)
        prefix = jnp.where(cnt >= K, cand, prefix)
    mask = bits >= prefix
    xrm_ref[...] = jnp.where(mask, xr_ref[...], 0.0)
    xim_ref[...] = jnp.where(mask, xi_ref[...], 0.0)


def _inv_kernel(grh_ref, grl_ref, gih_ref, gil_ref, xrm_ref, xim_ref, out_ref):
    bf16, f32 = jnp.bfloat16, jnp.float32
    xr = xrm_ref[...]
    xi = xim_ref[...]
    xrh = xr.astype(bf16)
    xrl = (xr - xrh.astype(f32)).astype(bf16)
    xih = xi.astype(bf16)
    xil = (xi - xih.astype(f32)).astype(bf16)
    dot = lambda a, b: jnp.dot(a, b, preferred_element_type=f32)
    out_ref[...] = (
        dot(grh_ref[...], xrh) + dot(grh_ref[...], xrl) + dot(grl_ref[...], xrh)
        + dot(gih_ref[...], xih) + dot(gih_ref[...], xil) + dot(gil_ref[...], xih))


def kernel(x):
    B, Tt, C = x.shape
    N = B * C
    xm = jnp.moveaxis(x, 0, 1).reshape(Tt, N)

    f32 = jnp.float32
    xr, xi, m2 = pl.pallas_call(
        _fwd_kernel,
        grid=(FP // _BM, N // _BN),
        in_specs=[
            pl.BlockSpec((_BM, T), lambda i, j: (i, 0)),
            pl.BlockSpec((_BM, T), lambda i, j: (i, 0)),
            pl.BlockSpec((T, _BN), lambda i, j: (0, j)),
        ],
        out_specs=[
            pl.BlockSpec((_BM, _BN), lambda i, j: (i, j)),
            pl.BlockSpec((_BM, _BN), lambda i, j: (i, j)),
            pl.BlockSpec((_BM, _BN), lambda i, j: (i, j)),
        ],
        out_shape=[jax.ShapeDtypeStruct((FP, N), f32)] * 3,
    )(jnp.asarray(_FC), jnp.asarray(_FS), xm)

    xrm, xim = pl.pallas_call(
        _mask_kernel,
        grid=(N // _BN,),
        in_specs=[pl.BlockSpec((FP, _BN), lambda j: (0, j))] * 3,
        out_specs=[pl.BlockSpec((FP, _BN), lambda j: (0, j))] * 2,
        out_shape=[jax.ShapeDtypeStruct((FP, N), f32)] * 2,
    )(m2, xr, xi)

    out = pl.pallas_call(
        _inv_kernel,
        grid=(Tt // _BT, N // _BN),
        in_specs=[
            pl.BlockSpec((_BT, FP), lambda i, j: (i, 0)),
            pl.BlockSpec((_BT, FP), lambda i, j: (i, 0)),
            pl.BlockSpec((_BT, FP), lambda i, j: (i, 0)),
            pl.BlockSpec((_BT, FP), lambda i, j: (i, 0)),
            pl.BlockSpec((FP, _BN), lambda i, j: (0, j)),
            pl.BlockSpec((FP, _BN), lambda i, j: (0, j)),
        ],
        out_specs=pl.BlockSpec((_BT, _BN), lambda i, j: (i, j)),
        out_shape=jax.ShapeDtypeStruct((Tt, N), f32),
    )(jnp.asarray(_GRH), jnp.asarray(_GRL), jnp.asarray(_GIH),
      jnp.asarray(_GIL), xrm, xim)

    return jnp.moveaxis(out.reshape(Tt, B, C), 0, 1)


# speedup vs baseline: 35.6987x; 1.0002x over previous
import numpy as np
import jax
import jax.numpy as jnp
from jax.experimental import pallas as pl
from jax.experimental.pallas import tpu as pltpu

T = 2048
FREQ = 1025
FP = 1152
K = 32
WIN = 64


def _build_consts():
    t = np.arange(T, dtype=np.float64)
    f = np.arange(FREQ, dtype=np.float64)
    ang = 2.0 * np.pi * np.outer(f, t) / T
    cos, sin = np.cos(ang), np.sin(ang)
    fc = np.zeros((FP, T)); fc[:FREQ] = cos
    fs = np.zeros((FP, T)); fs[:FREQ] = -sin
    w = np.full(FREQ, 2.0); w[0] = 1.0; w[-1] = 1.0
    cinv = cos.T * (w / T)
    sinv = -sin.T * (w / T)
    win = np.hamming(WIN)
    u = np.arange(T + WIN)
    src = np.where(u < 32, 31 - u, np.where(u < T + 32, u - 32, 2 * T + 31 - u))
    M = np.zeros((T, T))
    rows = np.arange(T)
    for j in range(WIN):
        M[rows, src[rows + j]] += win[j] / win.sum()
    gr = np.zeros((T, FP)); gr[:, :FREQ] = M @ cinv
    gi = np.zeros((T, FP)); gi[:, :FREQ] = M @ sinv
    return (fc.astype(np.float32), fs.astype(np.float32),
            gr.astype(np.float32), gi.astype(np.float32))


_FC, _FS, _GR, _GI = _build_consts()

_BM = 384
_BN = 512
_BT = 512


_PREC = jax.lax.Precision.HIGHEST


def _split_bf16(a32):
    import ml_dtypes
    hi = a32.astype(ml_dtypes.bfloat16)
    lo = (a32 - hi.astype(np.float32)).astype(ml_dtypes.bfloat16)
    return hi, lo


_GRH, _GRL = _split_bf16(_GR)
_GIH, _GIL = _split_bf16(_GI)


def _fwd_kernel(fc_ref, fs_ref, xm_ref, xr_ref, xi_ref, m2_ref):
    xm = xm_ref[...]
    xr = jnp.dot(fc_ref[...], xm, preferred_element_type=jnp.float32,
                 precision=_PREC)
    xi = jnp.dot(fs_ref[...], xm, preferred_element_type=jnp.float32,
                 precision=_PREC)
    xr_ref[...] = xr
    xi_ref[...] = xi
    m2_ref[...] = xr * xr + xi * xi


def _mask_kernel(m2_ref, xr_ref, xi_ref, xrm_ref, xim_ref):
    bits = pltpu.bitcast(m2_ref[...], jnp.int32)
    prefix = jnp.zeros((1, bits.shape[1]), jnp.int32)
    for b in range(30, -1, -1):
        cand = prefix | (1 << b)
        cnt = jnp.sum((bits >= cand).astype(jnp.int32), axis=0, keepdims=True)
        prefix = jnp.where(cnt >= K, cand, prefix)
    mask = bits >= prefix
    xrm_ref[...] = jnp.where(mask, xr_ref[...], 0.0)
    xim_ref[...] = jnp.where(mask, xi_ref[...], 0.0)


def _inv_kernel(gr_ref, gi_ref, xrm_ref, xim_ref, out_ref):
    bf16, f32 = jnp.bfloat16, jnp.float32
    xr = xrm_ref[...]
    xi = xim_ref[...]
    xrh = xr.astype(bf16)
    xrl = (xr - xrh.astype(f32)).astype(bf16)
    xih = xi.astype(bf16)
    xil = (xi - xih.astype(f32)).astype(bf16)
    gr = gr_ref[...]
    grh = gr.astype(bf16)
    grl = (gr - grh.astype(f32)).astype(bf16)
    gi = gi_ref[...]
    gih = gi.astype(bf16)
    gil = (gi - gih.astype(f32)).astype(bf16)
    dot = lambda a, b: jnp.dot(a, b, preferred_element_type=f32)
    out_ref[...] = (
        dot(grh, xrh) + dot(grh, xrl) + dot(grl, xrh)
        + dot(gih, xih) + dot(gih, xil) + dot(gil, xih))


def kernel(x):
    B, Tt, C = x.shape
    N = B * C
    xm = jnp.moveaxis(x, 0, 1).reshape(Tt, N)

    f32 = jnp.float32
    xr, xi, m2 = pl.pallas_call(
        _fwd_kernel,
        grid=(FP // _BM, N // _BN),
        in_specs=[
            pl.BlockSpec((_BM, T), lambda i, j: (i, 0)),
            pl.BlockSpec((_BM, T), lambda i, j: (i, 0)),
            pl.BlockSpec((T, _BN), lambda i, j: (0, j)),
        ],
        out_specs=[
            pl.BlockSpec((_BM, _BN), lambda i, j: (i, j)),
            pl.BlockSpec((_BM, _BN), lambda i, j: (i, j)),
            pl.BlockSpec((_BM, _BN), lambda i, j: (i, j)),
        ],
        out_shape=[jax.ShapeDtypeStruct((FP, N), f32)] * 3,
    )(jnp.asarray(_FC), jnp.asarray(_FS), xm)

    xrm, xim = pl.pallas_call(
        _mask_kernel,
        grid=(N // _BN,),
        in_specs=[pl.BlockSpec((FP, _BN), lambda j: (0, j))] * 3,
        out_specs=[pl.BlockSpec((FP, _BN), lambda j: (0, j))] * 2,
        out_shape=[jax.ShapeDtypeStruct((FP, N), f32)] * 2,
    )(m2, xr, xi)

    out = pl.pallas_call(
        _inv_kernel,
        grid=(Tt // _BT, N // _BN),
        in_specs=[
            pl.BlockSpec((_BT, FP), lambda i, j: (i, 0)),
            pl.BlockSpec((_BT, FP), lambda i, j: (i, 0)),
            pl.BlockSpec((FP, _BN), lambda i, j: (0, j)),
            pl.BlockSpec((FP, _BN), lambda i, j: (0, j)),
        ],
        out_specs=pl.BlockSpec((_BT, _BN), lambda i, j: (i, j)),
        out_shape=jax.ShapeDtypeStruct((Tt, N), f32),
    )(jnp.asarray(_GR), jnp.asarray(_GI), xrm, xim)

    return jnp.moveaxis(out.reshape(Tt, B, C), 0, 1)


# speedup vs baseline: 46.4015x; 1.2998x over previous
import numpy as np
import jax
import jax.numpy as jnp
from jax.experimental import pallas as pl
from jax.experimental.pallas import tpu as pltpu

T = 2048
FREQ = 1025
FP = 1152
K = 32
WIN = 64


def _build_consts():
    perm = np.full(FP, -1, dtype=np.int64)
    perm[:513] = 2 * np.arange(513)
    perm[576:1088] = 2 * np.arange(512) + 1
    valid = perm >= 0
    th = np.arange(T // 2, dtype=np.float64)
    ang_f = 2.0 * np.pi * np.outer(np.where(valid, perm, 0), th) / T
    fc = np.where(valid[:, None], np.cos(ang_f), 0.0)
    fs = np.where(valid[:, None], -np.sin(ang_f), 0.0)
    t = np.arange(T, dtype=np.float64)
    f = np.arange(FREQ, dtype=np.float64)
    ang = 2.0 * np.pi * np.outer(f, t) / T
    w = np.full(FREQ, 2.0); w[0] = 1.0; w[-1] = 1.0
    cinv = np.cos(ang).T * (w / T)
    sinv = -np.sin(ang).T * (w / T)
    win = np.hamming(WIN)
    u = np.arange(T + WIN)
    src = np.where(u < 32, 31 - u, np.where(u < T + 32, u - 32, 2 * T + 31 - u))
    M = np.zeros((T, T))
    rows = np.arange(T)
    for j in range(WIN):
        M[rows, src[rows + j]] += win[j] / win.sum()
    grf = M @ cinv
    gif = M @ sinv
    gr = np.zeros((T, FP)); gr[:, valid] = grf[:, perm[valid]]
    gi = np.zeros((T, FP)); gi[:, valid] = gif[:, perm[valid]]
    return (fc.astype(np.float32), fs.astype(np.float32),
            gr.astype(np.float32), gi.astype(np.float32))


_FC, _FS, _GR, _GI = _build_consts()

_BM = 576
_BN = 512
_BT = 512


_PREC = jax.lax.Precision.HIGHEST


def _split_bf16(a32):
    import ml_dtypes
    hi = a32.astype(ml_dtypes.bfloat16)
    lo = (a32 - hi.astype(np.float32)).astype(ml_dtypes.bfloat16)
    return hi, lo


_GRH, _GRL = _split_bf16(_GR)
_GIH, _GIL = _split_bf16(_GI)


def _fwd_kernel(fc_ref, fs_ref, xm_ref, xr_ref, xi_ref, m2_ref):
    sign = jnp.where(pl.program_id(0) == 0, 1.0, -1.0)
    y = xm_ref[: T // 2, :] + sign * xm_ref[T // 2:, :]
    xr = jnp.dot(fc_ref[...], y, preferred_element_type=jnp.float32,
                 precision=_PREC)
    xi = jnp.dot(fs_ref[...], y, preferred_element_type=jnp.float32,
                 precision=_PREC)
    xr_ref[...] = xr
    xi_ref[...] = xi
    m2_ref[...] = xr * xr + xi * xi


def _mask_kernel(m2_ref, xr_ref, xi_ref, xrm_ref, xim_ref):
    bits = pltpu.bitcast(m2_ref[...], jnp.int32)
    prefix = jnp.zeros((1, bits.shape[1]), jnp.int32)
    for b in range(30, -1, -1):
        cand = prefix | (1 << b)
        cnt = jnp.sum((bits >= cand).astype(jnp.int32), axis=0, keepdims=True)
        prefix = jnp.where(cnt >= K, cand, prefix)
    mask = bits >= prefix
    xrm_ref[...] = jnp.where(mask, xr_ref[...], 0.0)
    xim_ref[...] = jnp.where(mask, xi_ref[...], 0.0)


def _inv_kernel(gr_ref, gi_ref, xrm_ref, xim_ref, out_ref):
    bf16, f32 = jnp.bfloat16, jnp.float32
    xr = xrm_ref[...]
    xi = xim_ref[...]
    xrh = xr.astype(bf16)
    xrl = (xr - xrh.astype(f32)).astype(bf16)
    xih = xi.astype(bf16)
    xil = (xi - xih.astype(f32)).astype(bf16)
    gr = gr_ref[...]
    grh = gr.astype(bf16)
    grl = (gr - grh.astype(f32)).astype(bf16)
    gi = gi_ref[...]
    gih = gi.astype(bf16)
    gil = (gi - gih.astype(f32)).astype(bf16)
    dot = lambda a, b: jnp.dot(a, b, preferred_element_type=f32)
    out_ref[...] = (
        dot(grh, xrh) + dot(grh, xrl) + dot(grl, xrh)
        + dot(gih, xih) + dot(gih, xil) + dot(gil, xih))


def kernel(x):
    B, Tt, C = x.shape
    N = B * C
    xm = jnp.moveaxis(x, 0, 1).reshape(Tt, N)

    f32 = jnp.float32
    xr, xi, m2 = pl.pallas_call(
        _fwd_kernel,
        grid=(2, N // _BN),
        in_specs=[
            pl.BlockSpec((_BM, T // 2), lambda i, j: (i, 0)),
            pl.BlockSpec((_BM, T // 2), lambda i, j: (i, 0)),
            pl.BlockSpec((T, _BN), lambda i, j: (0, j)),
        ],
        out_specs=[
            pl.BlockSpec((_BM, _BN), lambda i, j: (i, j)),
            pl.BlockSpec((_BM, _BN), lambda i, j: (i, j)),
            pl.BlockSpec((_BM, _BN), lambda i, j: (i, j)),
        ],
        out_shape=[jax.ShapeDtypeStruct((FP, N), f32)] * 3,
    )(jnp.asarray(_FC), jnp.asarray(_FS), xm)

    xrm, xim = pl.pallas_call(
        _mask_kernel,
        grid=(N // _BN,),
        in_specs=[pl.BlockSpec((FP, _BN), lambda j: (0, j))] * 3,
        out_specs=[pl.BlockSpec((FP, _BN), lambda j: (0, j))] * 2,
        out_shape=[jax.ShapeDtypeStruct((FP, N), f32)] * 2,
    )(m2, xr, xi)

    out = pl.pallas_call(
        _inv_kernel,
        grid=(Tt // _BT, N // _BN),
        in_specs=[
            pl.BlockSpec((_BT, FP), lambda i, j: (i, 0)),
            pl.BlockSpec((_BT, FP), lambda i, j: (i, 0)),
            pl.BlockSpec((FP, _BN), lambda i, j: (0, j)),
            pl.BlockSpec((FP, _BN), lambda i, j: (0, j)),
        ],
        out_specs=pl.BlockSpec((_BT, _BN), lambda i, j: (i, j)),
        out_shape=jax.ShapeDtypeStruct((Tt, N), f32),
    )(jnp.asarray(_GR), jnp.asarray(_GI), xrm, xim)

    return jnp.moveaxis(out.reshape(Tt, B, C), 0, 1)


# speedup vs baseline: 47.8589x; 1.0314x over previous
import numpy as np
import jax
import jax.numpy as jnp
from jax.experimental import pallas as pl
from jax.experimental.pallas import tpu as pltpu

T = 2048
FREQ = 1025
FP = 1152
K = 32
WIN = 64


def _build_consts():
    perm = np.full(FP, -1, dtype=np.int64)
    perm[:513] = 2 * np.arange(513)
    perm[576:1088] = 2 * np.arange(512) + 1
    valid = perm >= 0
    th = np.arange(T // 2, dtype=np.float64)
    ang_f = 2.0 * np.pi * np.outer(np.where(valid, perm, 0), th) / T
    fc = np.where(valid[:, None], np.cos(ang_f), 0.0)
    fs = np.where(valid[:, None], -np.sin(ang_f), 0.0)
    t = np.arange(T, dtype=np.float64)
    f = np.arange(FREQ, dtype=np.float64)
    ang = 2.0 * np.pi * np.outer(f, t) / T
    w = np.full(FREQ, 2.0); w[0] = 1.0; w[-1] = 1.0
    cinv = np.cos(ang).T * (w / T)
    sinv = -np.sin(ang).T * (w / T)
    win = np.hamming(WIN)
    u = np.arange(T + WIN)
    src = np.where(u < 32, 31 - u, np.where(u < T + 32, u - 32, 2 * T + 31 - u))
    M = np.zeros((T, T))
    rows = np.arange(T)
    for j in range(WIN):
        M[rows, src[rows + j]] += win[j] / win.sum()
    grf = M @ cinv
    gif = M @ sinv
    gr = np.zeros((T, FP)); gr[:, valid] = grf[:, perm[valid]]
    gi = np.zeros((T, FP)); gi[:, valid] = gif[:, perm[valid]]
    return (fc.astype(np.float32), fs.astype(np.float32),
            gr.astype(np.float32), gi.astype(np.float32))


_FC, _FS, _GR, _GI = _build_consts()

_BM = 576
_BN = 512
_BT = 512


_PREC = jax.lax.Precision.HIGHEST


def _fwdmask_kernel(fc_ref, fs_ref, xm_ref, xrm_ref, xim_ref, bits_ref):
    f32 = jnp.float32
    top = xm_ref[: T // 2, :]
    bot = xm_ref[T // 2:, :]
    ye = top + bot
    yo = top - bot
    dot = lambda a, b: jnp.dot(a, b, preferred_element_type=f32,
                               precision=_PREC)
    h = FP // 2
    xr_e = dot(fc_ref[:h, :], ye)
    xr_o = dot(fc_ref[h:, :], yo)
    xi_e = dot(fs_ref[:h, :], ye)
    xi_o = dot(fs_ref[h:, :], yo)
    bits_ref[:h, :] = pltpu.bitcast(xr_e * xr_e + xi_e * xi_e, jnp.int32)
    bits_ref[h:, :] = pltpu.bitcast(xr_o * xr_o + xi_o * xi_o, jnp.int32)
    bits_e = bits_ref[:h, :]
    bits_o = bits_ref[h:, :]
    prefix = jnp.zeros((1, bits_e.shape[1]), jnp.int32)
    for b in range(30, -1, -1):
        cand = prefix | (1 << b)
        cnt = (jnp.sum((bits_e >= cand).astype(jnp.int32), axis=0, keepdims=True)
               + jnp.sum((bits_o >= cand).astype(jnp.int32), axis=0, keepdims=True))
        prefix = jnp.where(cnt >= K, cand, prefix)
    xrm_ref[:h, :] = xr_e * (bits_e >= prefix).astype(f32)
    xrm_ref[h:, :] = xr_o * (bits_o >= prefix).astype(f32)
    xim_ref[:h, :] = xi_e * (bits_e >= prefix).astype(f32)
    xim_ref[h:, :] = xi_o * (bits_o >= prefix).astype(f32)


def _inv_kernel(gr_ref, gi_ref, xrm_ref, xim_ref, out_ref):
    bf16, f32 = jnp.bfloat16, jnp.float32
    xr = xrm_ref[...]
    xi = xim_ref[...]
    xrh = xr.astype(bf16)
    xrl = (xr - xrh.astype(f32)).astype(bf16)
    xih = xi.astype(bf16)
    xil = (xi - xih.astype(f32)).astype(bf16)
    dot = lambda a, b: jnp.dot(a, b, preferred_element_type=f32)
    for ti in range(T // _BT):
        gr = gr_ref[pl.ds(ti * _BT, _BT), :]
        grh = gr.astype(bf16)
        grl = (gr - grh.astype(f32)).astype(bf16)
        gi = gi_ref[pl.ds(ti * _BT, _BT), :]
        gih = gi.astype(bf16)
        gil = (gi - gih.astype(f32)).astype(bf16)
        out_ref[pl.ds(ti * _BT, _BT), :] = (
            dot(grh, xrh) + dot(grh, xrl) + dot(grl, xrh)
            + dot(gih, xih) + dot(gih, xil) + dot(gil, xih))


def kernel(x):
    B, Tt, C = x.shape
    N = B * C
    xm = jnp.moveaxis(x, 0, 1).reshape(Tt, N)

    f32 = jnp.float32
    xrm, xim = pl.pallas_call(
        _fwdmask_kernel,
        grid=(N // _BN,),
        in_specs=[
            pl.BlockSpec((FP, T // 2), lambda j: (0, 0)),
            pl.BlockSpec((FP, T // 2), lambda j: (0, 0)),
            pl.BlockSpec((T, _BN), lambda j: (0, j)),
        ],
        out_specs=[
            pl.BlockSpec((FP, _BN), lambda j: (0, j)),
            pl.BlockSpec((FP, _BN), lambda j: (0, j)),
        ],
        out_shape=[jax.ShapeDtypeStruct((FP, N), f32)] * 2,
        scratch_shapes=[pltpu.VMEM((FP, _BN), jnp.int32)],
    )(jnp.asarray(_FC), jnp.asarray(_FS), xm)

    out = pl.pallas_call(
        _inv_kernel,
        grid=(N // _BN,),
        in_specs=[
            pl.BlockSpec((Tt, FP), lambda j: (0, 0)),
            pl.BlockSpec((Tt, FP), lambda j: (0, 0)),
            pl.BlockSpec((FP, _BN), lambda j: (0, j)),
            pl.BlockSpec((FP, _BN), lambda j: (0, j)),
        ],
        out_specs=pl.BlockSpec((Tt, _BN), lambda j: (0, j)),
        out_shape=jax.ShapeDtypeStruct((Tt, N), f32),
    )(jnp.asarray(_GR), jnp.asarray(_GI), xrm, xim)

    return jnp.moveaxis(out.reshape(Tt, B, C), 0, 1)
